# Initial kernel scaffold; baseline (speedup 1.0000x reference)
import jax
import jax.numpy as jnp
from jax.experimental import pallas as pl


def kernel(parameters, energies, transfer_matrix):
    raise NotImplementedError("write your pallas kernel here")


# speedup vs baseline: 1.0481x; 1.0481x over previous
import jax
import jax.numpy as jnp
from jax.experimental import pallas as pl
from jax.experimental.pallas import tpu as pltpu

N_CHANNELS = 4096
N_ENERGIES = 16384
BC = 256


def _body(params_ref, en_ref, tm_ref, out_ref, flux_ref):
    @pl.when(pl.program_id(0) == 0)
    def _():
        alpha = params_ref[0] + 1.2
        norm = params_ref[1]
        one_m_a = 1.0 - alpha
        e_low = en_ref[0:1, :]
        e_high = en_ref[1:2, :]
        flux_ref[...] = norm * (
            jnp.power(e_high, one_m_a) - jnp.power(e_low, one_m_a)
        ) / one_m_a

    flux = flux_ref[...]
    acc = jnp.sum(tm_ref[...] * flux, axis=1)
    out_ref[...] = jnp.maximum(acc, 1e-6)


def kernel(parameters, energies, transfer_matrix):
    out = pl.pallas_call(
        _body,
        grid=(N_CHANNELS // BC,),
        in_specs=[
            pl.BlockSpec(memory_space=pltpu.SMEM),
            pl.BlockSpec((2, N_ENERGIES), lambda i: (0, 0)),
            pl.BlockSpec((BC, N_ENERGIES), lambda i: (i, 0)),
        ],
        out_specs=pl.BlockSpec((BC,), lambda i: (i,)),
        out_shape=jax.ShapeDtypeStruct((N_CHANNELS,), jnp.float32),
        scratch_shapes=[pltpu.VMEM((1, N_ENERGIES), jnp.float32)],
    )(parameters, energies, transfer_matrix)
    return out


# speedup vs baseline: 1.0613x; 1.0125x over previous
import jax
import jax.numpy as jnp
from jax.experimental import pallas as pl
from jax.experimental.pallas import tpu as pltpu

N_CHANNELS = 4096
N_ENERGIES = 16384
BC = 128


def _body(params_ref, en_ref, tm_ref, out_ref, flux_ref):
    @pl.when(pl.program_id(0) == 0)
    def _():
        alpha = params_ref[0] + 1.2
        norm = params_ref[1]
        one_m_a = 1.0 - alpha
        e_low = en_ref[0:1, :]
        e_high = en_ref[1:2, :]
        flux_ref[...] = norm * (
            jnp.power(e_high, one_m_a) - jnp.power(e_low, one_m_a)
        ) / one_m_a

    flux = flux_ref[...]
    acc = jnp.sum(tm_ref[...] * flux, axis=1)
    out_ref[...] = jnp.maximum(acc, 1e-6)


def kernel(parameters, energies, transfer_matrix):
    out = pl.pallas_call(
        _body,
        grid=(N_CHANNELS // BC,),
        in_specs=[
            pl.BlockSpec(memory_space=pltpu.SMEM),
            pl.BlockSpec((2, N_ENERGIES), lambda i: (0, 0)),
            pl.BlockSpec((BC, N_ENERGIES), lambda i: (i, 0)),
        ],
        out_specs=pl.BlockSpec((BC,), lambda i: (i,)),
        out_shape=jax.ShapeDtypeStruct((N_CHANNELS,), jnp.float32),
        scratch_shapes=[pltpu.VMEM((1, N_ENERGIES), jnp.float32)],
    )(parameters, energies, transfer_matrix)
    return out
